# initial kernel scaffold (unmeasured)
import functools

import jax
import jax.numpy as jnp
from jax import lax
from jax.experimental import pallas as pl
from jax.experimental.pallas import tpu as pltpu

N_DEV = 4
B, SQ, H, D = 8, 8, 16, 128
SKV = 1024
SCALE = D ** -0.5


def _partial_body(q_ref, k_ref, v_ref, o_ref, l_ref):
    q = q_ref[0, :, 0, :]
    k = k_ref[0, :, 0, :]
    v = v_ref[0, :, 0, :]
    s = lax.dot_general(
        q, k, (((1,), (1,)), ((), ())), preferred_element_type=jnp.float32
    ) * SCALE
    p = jnp.exp(s)
    l_ref[...] = jnp.sum(p, axis=1, keepdims=True)[None]
    o = lax.dot_general(
        p, v, (((1,), (0,)), ((), ())), preferred_element_type=jnp.float32
    )
    o_ref[...] = o[None, :, None, :]


def _partials(Q, K, V):
    grid = (B, H)
    return pl.pallas_call(
        _partial_body,
        grid=grid,
        in_specs=[
            pl.BlockSpec((1, SQ, 1, D), lambda b, h: (b, 0, h, 0)),
            pl.BlockSpec((1, SKV, 1, D), lambda b, h: (b, 0, h, 0)),
            pl.BlockSpec((1, SKV, 1, D), lambda b, h: (b, 0, h, 0)),
        ],
        out_specs=[
            pl.BlockSpec((1, SQ, 1, D), lambda b, h: (b, 0, h, 0)),
            pl.BlockSpec((1, SQ, 1), lambda b, h: (b, 0, h)),
        ],
        out_shape=[
            jax.ShapeDtypeStruct((B, SQ, H, D), jnp.float32),
            jax.ShapeDtypeStruct((B, SQ, H), jnp.float32),
        ],
    )(Q, K, V)


def _allreduce_body(x_ref, out_ref, comm_ref, send_sems, recv_sems):
    my_pos = lax.axis_index("i")
    left = (my_pos - 1) % N_DEV
    right = (my_pos + 1) % N_DEV

    barrier_sem = pltpu.get_barrier_semaphore()
    for nbr in [left, right]:
        pl.semaphore_signal(
            barrier_sem, inc=1,
            device_id=(nbr,), device_id_type=pl.DeviceIdType.MESH,
        )
    pl.semaphore_wait(barrier_sem, 2)

    comm_ref[0] = x_ref[...]
    acc = x_ref[...]

    for h in range(N_DEV - 1):
        send_slot = h % 2
        recv_slot = (h + 1) % 2
        rdma = pltpu.make_async_remote_copy(
            src_ref=comm_ref.at[send_slot],
            dst_ref=comm_ref.at[recv_slot],
            send_sem=send_sems.at[send_slot],
            recv_sem=recv_sems.at[recv_slot],
            device_id=(right,),
            device_id_type=pl.DeviceIdType.MESH,
        )
        rdma.start()
        rdma.wait()
        acc = acc + comm_ref[recv_slot]

    out_ref[...] = acc


def _ring_allreduce(buf):
    rows, n = buf.shape
    return pl.pallas_call(
        _allreduce_body,
        out_shape=jax.ShapeDtypeStruct((rows, n), jnp.float32),
        in_specs=[pl.BlockSpec(memory_space=pltpu.VMEM)],
        out_specs=pl.BlockSpec(memory_space=pltpu.VMEM),
        scratch_shapes=[
            pltpu.VMEM((2, rows, n), jnp.float32),
            pltpu.SemaphoreType.DMA((2,)),
            pltpu.SemaphoreType.DMA((2,)),
        ],
        compiler_params=pltpu.CompilerParams(collective_id=0),
    )(buf)


def kernel(Q, K, V):
    o_part, l_part = _partials(Q, K, V)
    buf = jnp.concatenate(
        [o_part.reshape(B * SQ * H, D), l_part.reshape(B * SQ * H // D, D)],
        axis=0,
    )
    red = _ring_allreduce(buf)
    o_sum = red[: B * SQ * H].reshape(B, SQ, H, D)
    l_sum = red[B * SQ * H :].reshape(B, SQ, H)
    return o_sum / l_sum[..., None]


# baseline (device time: 273037 ns/iter reference)
import functools

import jax
import jax.numpy as jnp
from jax import lax
from jax.experimental import pallas as pl
from jax.experimental.pallas import tpu as pltpu

N_DEV = 4
B, SQ, H, D = 8, 8, 16, 128
SKV = 1024
SCALE = D ** -0.5


def _partial_body(q_ref, k_ref, v_ref, o_ref, l_ref):
    q = q_ref[0]
    k = k_ref[0]
    v = v_ref[0]
    s = lax.dot_general(
        q, k, (((1,), (1,)), ((), ())), preferred_element_type=jnp.float32
    ) * SCALE
    p = jnp.exp(s)
    l_ref[...] = jnp.sum(p, axis=1).reshape(1, 1, SQ, 1)
    o = lax.dot_general(
        p, v, (((1,), (0,)), ((), ())), preferred_element_type=jnp.float32
    )
    o_ref[0] = o


def _partials(Q, K, V):
    Q2 = Q.reshape(B, SQ, H * D)
    K2 = K.reshape(B, SKV, H * D)
    V2 = V.reshape(B, SKV, H * D)
    o, l = pl.pallas_call(
        _partial_body,
        grid=(B, H),
        in_specs=[
            pl.BlockSpec((1, SQ, D), lambda b, h: (b, 0, h)),
            pl.BlockSpec((1, SKV, D), lambda b, h: (b, 0, h)),
            pl.BlockSpec((1, SKV, D), lambda b, h: (b, 0, h)),
        ],
        out_specs=[
            pl.BlockSpec((1, SQ, D), lambda b, h: (b, 0, h)),
            pl.BlockSpec((1, 1, SQ, 1), lambda b, h: (b, h, 0, 0)),
        ],
        out_shape=[
            jax.ShapeDtypeStruct((B, SQ, H * D), jnp.float32),
            jax.ShapeDtypeStruct((B, H, SQ, 1), jnp.float32),
        ],
    )(Q2, K2, V2)
    return o, l


def _allreduce_body(x_ref, out_ref, comm_ref, send_sems, recv_sems):
    my_pos = lax.axis_index("i")
    left = (my_pos - 1) % N_DEV
    right = (my_pos + 1) % N_DEV

    barrier_sem = pltpu.get_barrier_semaphore()
    for nbr in [left, right]:
        pl.semaphore_signal(
            barrier_sem, inc=1,
            device_id=(nbr,), device_id_type=pl.DeviceIdType.MESH,
        )
    pl.semaphore_wait(barrier_sem, 2)

    comm_ref[0] = x_ref[...]
    acc = x_ref[...]

    for h in range(N_DEV - 1):
        send_slot = h % 2
        recv_slot = (h + 1) % 2
        rdma = pltpu.make_async_remote_copy(
            src_ref=comm_ref.at[send_slot],
            dst_ref=comm_ref.at[recv_slot],
            send_sem=send_sems.at[send_slot],
            recv_sem=recv_sems.at[recv_slot],
            device_id=(right,),
            device_id_type=pl.DeviceIdType.MESH,
        )
        rdma.start()
        rdma.wait()
        acc = acc + comm_ref[recv_slot]

    out_ref[...] = acc


def _ring_allreduce(buf):
    rows, n = buf.shape
    return pl.pallas_call(
        _allreduce_body,
        out_shape=jax.ShapeDtypeStruct((rows, n), jnp.float32),
        in_specs=[pl.BlockSpec(memory_space=pltpu.VMEM)],
        out_specs=pl.BlockSpec(memory_space=pltpu.VMEM),
        scratch_shapes=[
            pltpu.VMEM((2, rows, n), jnp.float32),
            pltpu.SemaphoreType.DMA((2,)),
            pltpu.SemaphoreType.DMA((2,)),
        ],
        compiler_params=pltpu.CompilerParams(collective_id=0),
    )(buf)


def kernel(Q, K, V):
    o_part, l_part = _partials(Q, K, V)
    buf = jnp.concatenate(
        [o_part.reshape(B * SQ * H, D), l_part.reshape(B * H * SQ // D, D)],
        axis=0,
    )
    red = _ring_allreduce(buf)
    o_sum = red[: B * SQ * H].reshape(B, SQ, H, D)
    l_sum = red[B * SQ * H :].reshape(B, H, SQ).transpose(0, 2, 1)
    return o_sum / l_sum[..., None]


# device time: 80223 ns/iter; 3.4035x vs baseline; 3.4035x over previous
import functools

import jax
import jax.numpy as jnp
from jax import lax
from jax.experimental import pallas as pl
from jax.experimental.pallas import tpu as pltpu

N_DEV = 4
B, SQ, H, D = 8, 8, 16, 128
SKV = 1024
SCALE = D ** -0.5


def _partial_body(q_ref, k_ref, v_ref, o_ref, l_ref):
    q = q_ref[0]
    k = k_ref[0]
    v = v_ref[0]
    s = lax.dot_general(
        q, k, (((1,), (1,)), ((), ())), preferred_element_type=jnp.float32
    ) * SCALE
    p = jnp.exp(s)
    l_ref[...] = jnp.sum(p, axis=1).reshape(1, 1, SQ, 1)
    o = lax.dot_general(
        p, v, (((1,), (0,)), ((), ())), preferred_element_type=jnp.float32
    )
    o_ref[0] = o


def _partials(Q, K, V):
    Q2 = Q.reshape(B, SQ, H * D)
    K2 = K.reshape(B, SKV, H * D)
    V2 = V.reshape(B, SKV, H * D)
    o, l = pl.pallas_call(
        _partial_body,
        grid=(B, H),
        in_specs=[
            pl.BlockSpec((1, SQ, D), lambda b, h: (b, 0, h)),
            pl.BlockSpec((1, SKV, D), lambda b, h: (b, 0, h)),
            pl.BlockSpec((1, SKV, D), lambda b, h: (b, 0, h)),
        ],
        out_specs=[
            pl.BlockSpec((1, SQ, D), lambda b, h: (b, 0, h)),
            pl.BlockSpec((1, 1, SQ, 1), lambda b, h: (b, h, 0, 0)),
        ],
        out_shape=[
            jax.ShapeDtypeStruct((B, SQ, H * D), jnp.float32),
            jax.ShapeDtypeStruct((B, H, SQ, 1), jnp.float32),
        ],
    )(Q2, K2, V2)
    return o, l


def _mk_partials_pre(body):
    def run(Q2, K2, V2):
        return pl.pallas_call(
            body,
            grid=(B, H),
            in_specs=[
                pl.BlockSpec((1, SQ, D), lambda b, h: (b, 0, h)),
                pl.BlockSpec((1, SKV, D), lambda b, h: (b, 0, h)),
                pl.BlockSpec((1, SKV, D), lambda b, h: (b, 0, h)),
            ],
            out_specs=[
                pl.BlockSpec((1, SQ, D), lambda b, h: (b, 0, h)),
                pl.BlockSpec((1, 1, SQ, 1), lambda b, h: (b, h, 0, 0)),
            ],
            out_shape=[
                jax.ShapeDtypeStruct((B, SQ, H * D), jnp.float32),
                jax.ShapeDtypeStruct((B, H, SQ, 1), jnp.float32),
            ],
        )(Q2, K2, V2)
    return run


def _partial_body_bf16(q_ref, k_ref, v_ref, o_ref, l_ref):
    q = q_ref[0].astype(jnp.bfloat16)
    k = k_ref[0].astype(jnp.bfloat16)
    v = v_ref[0].astype(jnp.bfloat16)
    s = lax.dot_general(
        q, k, (((1,), (1,)), ((), ())), preferred_element_type=jnp.float32
    ) * SCALE
    p = jnp.exp(s)
    l_ref[...] = jnp.sum(p, axis=1).reshape(1, 1, SQ, 1)
    o = lax.dot_general(
        p.astype(jnp.bfloat16), v,
        (((1,), (0,)), ((), ())), preferred_element_type=jnp.float32,
    )
    o_ref[0] = o


def _partial_body_kmajor(q_ref, k_ref, v_ref, o_ref, l_ref):
    q = q_ref[0]
    k = k_ref[0]
    v = v_ref[0]
    st = lax.dot_general(
        k, q, (((1,), (1,)), ((), ())), preferred_element_type=jnp.float32
    ) * SCALE
    pt = jnp.exp(st)
    l_ref[...] = jnp.sum(pt, axis=0).reshape(1, 1, SQ, 1)
    o = lax.dot_general(
        pt, v, (((0,), (0,)), ((), ())), preferred_element_type=jnp.float32
    )
    o_ref[0] = o


def _bd_body(qt_ref, k_ref, v_ref, o_ref, l_ref):
    kvc = pl.program_id(1)
    k = k_ref[0].astype(jnp.bfloat16)
    v = v_ref[0].astype(jnp.bfloat16)
    qt = qt_ref[0].astype(jnp.bfloat16)
    g = lax.dot_general(
        k, qt, (((1,), (0,)), ((), ())), preferred_element_type=jnp.float32
    )
    rh = lax.broadcasted_iota(jnp.int32, g.shape, 0) % H
    ch = lax.broadcasted_iota(jnp.int32, g.shape, 1) % H
    p = jnp.where(rh == ch, jnp.exp(g), 0.0)
    lsum = jnp.sum(p, axis=0, keepdims=True)
    o = lax.dot_general(
        p.astype(jnp.bfloat16), v,
        (((0,), (0,)), ((), ())), preferred_element_type=jnp.float32,
    )

    @pl.when(kvc == 0)
    def _():
        o_ref[0] = o
        l_ref[0] = lsum

    @pl.when(kvc != 0)
    def _():
        o_ref[0] += o
        l_ref[0] += lsum


_BD_NCHUNK = 4


def _partials_bd(Q, K, V):
    Kf = K.reshape(B, SKV * H, D)
    Vf = V.reshape(B, SKV * H, D)
    ch = SKV * H // _BD_NCHUNK
    Qt = (Q * SCALE).transpose(0, 3, 1, 2).reshape(B, D, SQ * H)
    o, l = pl.pallas_call(
        _bd_body,
        grid=(B, _BD_NCHUNK),
        in_specs=[
            pl.BlockSpec((1, D, SQ * H), lambda b, c: (b, 0, 0)),
            pl.BlockSpec((1, ch, D), lambda b, c: (b, c, 0)),
            pl.BlockSpec((1, ch, D), lambda b, c: (b, c, 0)),
        ],
        out_specs=[
            pl.BlockSpec((1, SQ * H, D), lambda b, c: (b, 0, 0)),
            pl.BlockSpec((1, 1, SQ * H), lambda b, c: (b, 0, 0)),
        ],
        out_shape=[
            jax.ShapeDtypeStruct((B, SQ * H, D), jnp.float32),
            jax.ShapeDtypeStruct((B, 1, SQ * H), jnp.float32),
        ],
    )(Qt, Kf, Vf)
    return o, l


def _mk_partials_hblock(nh):
    def body(q_ref, k_ref, v_ref, o_ref, l_ref):
        for hi in range(nh):
            sl = slice(hi * D, (hi + 1) * D)
            q = q_ref[0, :, sl]
            k = k_ref[0, :, sl]
            v = v_ref[0, :, sl]
            s = lax.dot_general(
                q, k, (((1,), (1,)), ((), ())),
                preferred_element_type=jnp.float32,
            ) * SCALE
            p = jnp.exp(s)
            l_ref[0, hi] = jnp.sum(p, axis=1, keepdims=True)
            o_ref[0, :, sl] = lax.dot_general(
                p, v, (((1,), (0,)), ((), ())),
                preferred_element_type=jnp.float32,
            )

    def run(Q2, K2, V2):
        return pl.pallas_call(
            body,
            grid=(B, H // nh),
            in_specs=[
                pl.BlockSpec((1, SQ, nh * D), lambda b, h: (b, 0, h)),
                pl.BlockSpec((1, SKV, nh * D), lambda b, h: (b, 0, h)),
                pl.BlockSpec((1, SKV, nh * D), lambda b, h: (b, 0, h)),
            ],
            out_specs=[
                pl.BlockSpec((1, SQ, nh * D), lambda b, h: (b, 0, h)),
                pl.BlockSpec((1, nh, SQ, 1), lambda b, h: (b, h, 0, 0)),
            ],
            out_shape=[
                jax.ShapeDtypeStruct((B, SQ, H * D), jnp.float32),
                jax.ShapeDtypeStruct((B, H, SQ, 1), jnp.float32),
            ],
        )(Q2, K2, V2)
    return run


_partials_pre_h4 = _mk_partials_hblock(4)
_partials_pre_h8 = _mk_partials_hblock(8)
_partials_pre_h16 = _mk_partials_hblock(16)

_partials_pre = _mk_partials_pre(_partial_body)
_partials_pre_bf16 = _mk_partials_pre(_partial_body_bf16)
_partials_pre_kmajor = _mk_partials_pre(_partial_body_kmajor)


def _allreduce_body(x_ref, out_ref, comm_ref, send_sems, recv_sems):
    my_pos = lax.axis_index("i")
    left = (my_pos - 1) % N_DEV
    right = (my_pos + 1) % N_DEV

    barrier_sem = pltpu.get_barrier_semaphore()
    for nbr in [left, right]:
        pl.semaphore_signal(
            barrier_sem, inc=1,
            device_id=(nbr,), device_id_type=pl.DeviceIdType.MESH,
        )
    pl.semaphore_wait(barrier_sem, 2)

    comm_ref[0] = x_ref[...]
    acc = x_ref[...]

    for h in range(N_DEV - 1):
        send_slot = h % 2
        recv_slot = (h + 1) % 2
        rdma = pltpu.make_async_remote_copy(
            src_ref=comm_ref.at[send_slot],
            dst_ref=comm_ref.at[recv_slot],
            send_sem=send_sems.at[send_slot],
            recv_sem=recv_sems.at[recv_slot],
            device_id=(right,),
            device_id_type=pl.DeviceIdType.MESH,
        )
        rdma.start()
        rdma.wait()
        acc = acc + comm_ref[recv_slot]

    out_ref[...] = acc


def _ring_allreduce(buf):
    rows, n = buf.shape
    return pl.pallas_call(
        _allreduce_body,
        out_shape=jax.ShapeDtypeStruct((rows, n), jnp.float32),
        in_specs=[pl.BlockSpec(memory_space=pltpu.VMEM)],
        out_specs=pl.BlockSpec(memory_space=pltpu.VMEM),
        scratch_shapes=[
            pltpu.VMEM((2, rows, n), jnp.float32),
            pltpu.SemaphoreType.DMA((2,)),
            pltpu.SemaphoreType.DMA((2,)),
        ],
        compiler_params=pltpu.CompilerParams(collective_id=0),
    )(buf)


def kernel(Q, K, V):
    o_part, l_part = _partials_bd(Q, K, V)
    buf = jnp.concatenate(
        [o_part.reshape(B * SQ * H, D), l_part.reshape(B, SQ * H)],
        axis=0,
    )
    red = _ring_allreduce(buf)
    o_sum = red[: B * SQ * H].reshape(B, SQ, H, D)
    l_sum = red[B * SQ * H :].reshape(B, SQ, H)
    return o_sum / l_sum[..., None]


# device time: 72803 ns/iter; 3.7504x vs baseline; 1.1019x over previous
import functools

import jax
import jax.numpy as jnp
from jax import lax
from jax.experimental import pallas as pl
from jax.experimental.pallas import tpu as pltpu

N_DEV = 4
B, SQ, H, D = 8, 8, 16, 128
SKV = 1024
SCALE = D ** -0.5


def _partial_body(q_ref, k_ref, v_ref, o_ref, l_ref):
    q = q_ref[0]
    k = k_ref[0]
    v = v_ref[0]
    s = lax.dot_general(
        q, k, (((1,), (1,)), ((), ())), preferred_element_type=jnp.float32
    ) * SCALE
    p = jnp.exp(s)
    l_ref[...] = jnp.sum(p, axis=1).reshape(1, 1, SQ, 1)
    o = lax.dot_general(
        p, v, (((1,), (0,)), ((), ())), preferred_element_type=jnp.float32
    )
    o_ref[0] = o


def _partials(Q, K, V):
    Q2 = Q.reshape(B, SQ, H * D)
    K2 = K.reshape(B, SKV, H * D)
    V2 = V.reshape(B, SKV, H * D)
    o, l = pl.pallas_call(
        _partial_body,
        grid=(B, H),
        in_specs=[
            pl.BlockSpec((1, SQ, D), lambda b, h: (b, 0, h)),
            pl.BlockSpec((1, SKV, D), lambda b, h: (b, 0, h)),
            pl.BlockSpec((1, SKV, D), lambda b, h: (b, 0, h)),
        ],
        out_specs=[
            pl.BlockSpec((1, SQ, D), lambda b, h: (b, 0, h)),
            pl.BlockSpec((1, 1, SQ, 1), lambda b, h: (b, h, 0, 0)),
        ],
        out_shape=[
            jax.ShapeDtypeStruct((B, SQ, H * D), jnp.float32),
            jax.ShapeDtypeStruct((B, H, SQ, 1), jnp.float32),
        ],
    )(Q2, K2, V2)
    return o, l


def _mk_partials_pre(body):
    def run(Q2, K2, V2):
        return pl.pallas_call(
            body,
            grid=(B, H),
            in_specs=[
                pl.BlockSpec((1, SQ, D), lambda b, h: (b, 0, h)),
                pl.BlockSpec((1, SKV, D), lambda b, h: (b, 0, h)),
                pl.BlockSpec((1, SKV, D), lambda b, h: (b, 0, h)),
            ],
            out_specs=[
                pl.BlockSpec((1, SQ, D), lambda b, h: (b, 0, h)),
                pl.BlockSpec((1, 1, SQ, 1), lambda b, h: (b, h, 0, 0)),
            ],
            out_shape=[
                jax.ShapeDtypeStruct((B, SQ, H * D), jnp.float32),
                jax.ShapeDtypeStruct((B, H, SQ, 1), jnp.float32),
            ],
        )(Q2, K2, V2)
    return run


def _partial_body_bf16(q_ref, k_ref, v_ref, o_ref, l_ref):
    q = q_ref[0].astype(jnp.bfloat16)
    k = k_ref[0].astype(jnp.bfloat16)
    v = v_ref[0].astype(jnp.bfloat16)
    s = lax.dot_general(
        q, k, (((1,), (1,)), ((), ())), preferred_element_type=jnp.float32
    ) * SCALE
    p = jnp.exp(s)
    l_ref[...] = jnp.sum(p, axis=1).reshape(1, 1, SQ, 1)
    o = lax.dot_general(
        p.astype(jnp.bfloat16), v,
        (((1,), (0,)), ((), ())), preferred_element_type=jnp.float32,
    )
    o_ref[0] = o


def _partial_body_kmajor(q_ref, k_ref, v_ref, o_ref, l_ref):
    q = q_ref[0]
    k = k_ref[0]
    v = v_ref[0]
    st = lax.dot_general(
        k, q, (((1,), (1,)), ((), ())), preferred_element_type=jnp.float32
    ) * SCALE
    pt = jnp.exp(st)
    l_ref[...] = jnp.sum(pt, axis=0).reshape(1, 1, SQ, 1)
    o = lax.dot_general(
        pt, v, (((0,), (0,)), ((), ())), preferred_element_type=jnp.float32
    )
    o_ref[0] = o


def _bd_body(qt_ref, k_ref, v_ref, o_ref, l_ref):
    kvc = pl.program_id(1)
    k = k_ref[0].astype(jnp.bfloat16)
    v = v_ref[0].astype(jnp.bfloat16)
    qt = qt_ref[0].astype(jnp.bfloat16)
    g = lax.dot_general(
        k, qt, (((1,), (0,)), ((), ())), preferred_element_type=jnp.float32
    )
    rh = lax.broadcasted_iota(jnp.int32, g.shape, 0) % H
    ch = lax.broadcasted_iota(jnp.int32, g.shape, 1) % H
    p = jnp.where(rh == ch, jnp.exp(g), 0.0)
    lsum = jnp.sum(p, axis=0, keepdims=True)
    o = lax.dot_general(
        p.astype(jnp.bfloat16), v,
        (((0,), (0,)), ((), ())), preferred_element_type=jnp.float32,
    )

    @pl.when(kvc == 0)
    def _():
        o_ref[0] = o
        l_ref[0] = lsum

    @pl.when(kvc != 0)
    def _():
        o_ref[0] += o
        l_ref[0] += lsum


_BD_NCHUNK = 4


def _partials_bd(Q, K, V):
    Kf = K.reshape(B, SKV * H, D)
    Vf = V.reshape(B, SKV * H, D)
    ch = SKV * H // _BD_NCHUNK
    Qt = (Q * SCALE).transpose(0, 3, 1, 2).reshape(B, D, SQ * H)
    o, l = pl.pallas_call(
        _bd_body,
        grid=(B, _BD_NCHUNK),
        in_specs=[
            pl.BlockSpec((1, D, SQ * H), lambda b, c: (b, 0, 0)),
            pl.BlockSpec((1, ch, D), lambda b, c: (b, c, 0)),
            pl.BlockSpec((1, ch, D), lambda b, c: (b, c, 0)),
        ],
        out_specs=[
            pl.BlockSpec((1, SQ * H, D), lambda b, c: (b, 0, 0)),
            pl.BlockSpec((1, 1, SQ * H), lambda b, c: (b, 0, 0)),
        ],
        out_shape=[
            jax.ShapeDtypeStruct((B, SQ * H, D), jnp.float32),
            jax.ShapeDtypeStruct((B, 1, SQ * H), jnp.float32),
        ],
    )(Qt, Kf, Vf)
    return o, l


def _mk_partials_hblock(nh):
    def body(q_ref, k_ref, v_ref, o_ref, l_ref):
        for hi in range(nh):
            sl = slice(hi * D, (hi + 1) * D)
            q = q_ref[0, :, sl]
            k = k_ref[0, :, sl]
            v = v_ref[0, :, sl]
            s = lax.dot_general(
                q, k, (((1,), (1,)), ((), ())),
                preferred_element_type=jnp.float32,
            ) * SCALE
            p = jnp.exp(s)
            l_ref[0, hi] = jnp.sum(p, axis=1, keepdims=True)
            o_ref[0, :, sl] = lax.dot_general(
                p, v, (((1,), (0,)), ((), ())),
                preferred_element_type=jnp.float32,
            )

    def run(Q2, K2, V2):
        return pl.pallas_call(
            body,
            grid=(B, H // nh),
            in_specs=[
                pl.BlockSpec((1, SQ, nh * D), lambda b, h: (b, 0, h)),
                pl.BlockSpec((1, SKV, nh * D), lambda b, h: (b, 0, h)),
                pl.BlockSpec((1, SKV, nh * D), lambda b, h: (b, 0, h)),
            ],
            out_specs=[
                pl.BlockSpec((1, SQ, nh * D), lambda b, h: (b, 0, h)),
                pl.BlockSpec((1, nh, SQ, 1), lambda b, h: (b, h, 0, 0)),
            ],
            out_shape=[
                jax.ShapeDtypeStruct((B, SQ, H * D), jnp.float32),
                jax.ShapeDtypeStruct((B, H, SQ, 1), jnp.float32),
            ],
        )(Q2, K2, V2)
    return run


_partials_pre_h4 = _mk_partials_hblock(4)
_partials_pre_h8 = _mk_partials_hblock(8)
_partials_pre_h16 = _mk_partials_hblock(16)

_partials_pre = _mk_partials_pre(_partial_body)
_partials_pre_bf16 = _mk_partials_pre(_partial_body_bf16)
_partials_pre_kmajor = _mk_partials_pre(_partial_body_kmajor)


def _allreduce_body(x_ref, out_ref, comm_ref, send_sems, recv_sems):
    my_pos = lax.axis_index("i")
    left = (my_pos - 1) % N_DEV
    right = (my_pos + 1) % N_DEV

    barrier_sem = pltpu.get_barrier_semaphore()
    for nbr in [left, right]:
        pl.semaphore_signal(
            barrier_sem, inc=1,
            device_id=(nbr,), device_id_type=pl.DeviceIdType.MESH,
        )
    pl.semaphore_wait(barrier_sem, 2)

    comm_ref[0] = x_ref[...]
    acc = x_ref[...].astype(jnp.float32)

    for h in range(N_DEV - 1):
        send_slot = h % 2
        recv_slot = (h + 1) % 2
        rdma = pltpu.make_async_remote_copy(
            src_ref=comm_ref.at[send_slot],
            dst_ref=comm_ref.at[recv_slot],
            send_sem=send_sems.at[send_slot],
            recv_sem=recv_sems.at[recv_slot],
            device_id=(right,),
            device_id_type=pl.DeviceIdType.MESH,
        )
        rdma.start()
        rdma.wait()
        acc = acc + comm_ref[recv_slot].astype(jnp.float32)

    out_ref[...] = acc


def _ring_allreduce(buf):
    rows, n = buf.shape
    return pl.pallas_call(
        _allreduce_body,
        out_shape=jax.ShapeDtypeStruct((rows, n), jnp.float32),
        in_specs=[pl.BlockSpec(memory_space=pltpu.VMEM)],
        out_specs=pl.BlockSpec(memory_space=pltpu.VMEM),
        scratch_shapes=[
            pltpu.VMEM((2, rows, n), jnp.bfloat16),
            pltpu.SemaphoreType.DMA((2,)),
            pltpu.SemaphoreType.DMA((2,)),
        ],
        compiler_params=pltpu.CompilerParams(collective_id=0),
    )(buf)


def kernel(Q, K, V):
    o_part, l_part = _partials_bd(Q, K, V)
    buf = jnp.concatenate(
        [o_part.reshape(B * SQ * H, D), l_part.reshape(B, SQ * H)],
        axis=0,
    ).astype(jnp.bfloat16)
    red = _ring_allreduce(buf)
    o_sum = red[: B * SQ * H].reshape(B, SQ, H, D)
    l_sum = red[B * SQ * H :].reshape(B, SQ, H)
    return o_sum / l_sum[..., None]


# device time: 64068 ns/iter; 4.2617x vs baseline; 1.1363x over previous
import functools

import jax
import jax.numpy as jnp
from jax import lax
from jax.experimental import pallas as pl
from jax.experimental.pallas import tpu as pltpu

N_DEV = 4
B, SQ, H, D = 8, 8, 16, 128
SKV = 1024
SCALE = D ** -0.5


def _partial_body(q_ref, k_ref, v_ref, o_ref, l_ref):
    q = q_ref[0]
    k = k_ref[0]
    v = v_ref[0]
    s = lax.dot_general(
        q, k, (((1,), (1,)), ((), ())), preferred_element_type=jnp.float32
    ) * SCALE
    p = jnp.exp(s)
    l_ref[...] = jnp.sum(p, axis=1).reshape(1, 1, SQ, 1)
    o = lax.dot_general(
        p, v, (((1,), (0,)), ((), ())), preferred_element_type=jnp.float32
    )
    o_ref[0] = o


def _partials(Q, K, V):
    Q2 = Q.reshape(B, SQ, H * D)
    K2 = K.reshape(B, SKV, H * D)
    V2 = V.reshape(B, SKV, H * D)
    o, l = pl.pallas_call(
        _partial_body,
        grid=(B, H),
        in_specs=[
            pl.BlockSpec((1, SQ, D), lambda b, h: (b, 0, h)),
            pl.BlockSpec((1, SKV, D), lambda b, h: (b, 0, h)),
            pl.BlockSpec((1, SKV, D), lambda b, h: (b, 0, h)),
        ],
        out_specs=[
            pl.BlockSpec((1, SQ, D), lambda b, h: (b, 0, h)),
            pl.BlockSpec((1, 1, SQ, 1), lambda b, h: (b, h, 0, 0)),
        ],
        out_shape=[
            jax.ShapeDtypeStruct((B, SQ, H * D), jnp.float32),
            jax.ShapeDtypeStruct((B, H, SQ, 1), jnp.float32),
        ],
    )(Q2, K2, V2)
    return o, l


def _mk_partials_pre(body):
    def run(Q2, K2, V2):
        return pl.pallas_call(
            body,
            grid=(B, H),
            in_specs=[
                pl.BlockSpec((1, SQ, D), lambda b, h: (b, 0, h)),
                pl.BlockSpec((1, SKV, D), lambda b, h: (b, 0, h)),
                pl.BlockSpec((1, SKV, D), lambda b, h: (b, 0, h)),
            ],
            out_specs=[
                pl.BlockSpec((1, SQ, D), lambda b, h: (b, 0, h)),
                pl.BlockSpec((1, 1, SQ, 1), lambda b, h: (b, h, 0, 0)),
            ],
            out_shape=[
                jax.ShapeDtypeStruct((B, SQ, H * D), jnp.float32),
                jax.ShapeDtypeStruct((B, H, SQ, 1), jnp.float32),
            ],
        )(Q2, K2, V2)
    return run


def _partial_body_bf16(q_ref, k_ref, v_ref, o_ref, l_ref):
    q = q_ref[0].astype(jnp.bfloat16)
    k = k_ref[0].astype(jnp.bfloat16)
    v = v_ref[0].astype(jnp.bfloat16)
    s = lax.dot_general(
        q, k, (((1,), (1,)), ((), ())), preferred_element_type=jnp.float32
    ) * SCALE
    p = jnp.exp(s)
    l_ref[...] = jnp.sum(p, axis=1).reshape(1, 1, SQ, 1)
    o = lax.dot_general(
        p.astype(jnp.bfloat16), v,
        (((1,), (0,)), ((), ())), preferred_element_type=jnp.float32,
    )
    o_ref[0] = o


def _partial_body_kmajor(q_ref, k_ref, v_ref, o_ref, l_ref):
    q = q_ref[0]
    k = k_ref[0]
    v = v_ref[0]
    st = lax.dot_general(
        k, q, (((1,), (1,)), ((), ())), preferred_element_type=jnp.float32
    ) * SCALE
    pt = jnp.exp(st)
    l_ref[...] = jnp.sum(pt, axis=0).reshape(1, 1, SQ, 1)
    o = lax.dot_general(
        pt, v, (((0,), (0,)), ((), ())), preferred_element_type=jnp.float32
    )
    o_ref[0] = o


def _bd_body(qt_ref, k_ref, v_ref, o_ref, l_ref):
    kvc = pl.program_id(1)
    k = k_ref[0].astype(jnp.bfloat16)
    v = v_ref[0].astype(jnp.bfloat16)
    qt = qt_ref[0].astype(jnp.bfloat16)
    g = lax.dot_general(
        k, qt, (((1,), (0,)), ((), ())), preferred_element_type=jnp.float32
    )
    rh = lax.broadcasted_iota(jnp.int32, g.shape, 0) % H
    ch = lax.broadcasted_iota(jnp.int32, g.shape, 1) % H
    p = jnp.where(rh == ch, jnp.exp(g), 0.0)
    lsum = jnp.sum(p, axis=0, keepdims=True)
    o = lax.dot_general(
        p.astype(jnp.bfloat16), v,
        (((0,), (0,)), ((), ())), preferred_element_type=jnp.float32,
    )

    @pl.when(kvc == 0)
    def _():
        o_ref[0] = o
        l_ref[0] = lsum

    @pl.when(kvc != 0)
    def _():
        o_ref[0] += o
        l_ref[0] += lsum


_BD_NCHUNK = 4


def _partials_bd(Q, K, V):
    Kf = K.reshape(B, SKV * H, D)
    Vf = V.reshape(B, SKV * H, D)
    ch = SKV * H // _BD_NCHUNK
    Qt = (Q * SCALE).transpose(0, 3, 1, 2).reshape(B, D, SQ * H)
    o, l = pl.pallas_call(
        _bd_body,
        grid=(B, _BD_NCHUNK),
        in_specs=[
            pl.BlockSpec((1, D, SQ * H), lambda b, c: (b, 0, 0)),
            pl.BlockSpec((1, ch, D), lambda b, c: (b, c, 0)),
            pl.BlockSpec((1, ch, D), lambda b, c: (b, c, 0)),
        ],
        out_specs=[
            pl.BlockSpec((1, SQ * H, D), lambda b, c: (b, 0, 0)),
            pl.BlockSpec((1, 1, SQ * H), lambda b, c: (b, 0, 0)),
        ],
        out_shape=[
            jax.ShapeDtypeStruct((B, SQ * H, D), jnp.float32),
            jax.ShapeDtypeStruct((B, 1, SQ * H), jnp.float32),
        ],
    )(Qt, Kf, Vf)
    return o, l


def _mk_partials_hblock(nh):
    def body(q_ref, k_ref, v_ref, o_ref, l_ref):
        for hi in range(nh):
            sl = slice(hi * D, (hi + 1) * D)
            q = q_ref[0, :, sl]
            k = k_ref[0, :, sl]
            v = v_ref[0, :, sl]
            s = lax.dot_general(
                q, k, (((1,), (1,)), ((), ())),
                preferred_element_type=jnp.float32,
            ) * SCALE
            p = jnp.exp(s)
            l_ref[0, hi] = jnp.sum(p, axis=1, keepdims=True)
            o_ref[0, :, sl] = lax.dot_general(
                p, v, (((1,), (0,)), ((), ())),
                preferred_element_type=jnp.float32,
            )

    def run(Q2, K2, V2):
        return pl.pallas_call(
            body,
            grid=(B, H // nh),
            in_specs=[
                pl.BlockSpec((1, SQ, nh * D), lambda b, h: (b, 0, h)),
                pl.BlockSpec((1, SKV, nh * D), lambda b, h: (b, 0, h)),
                pl.BlockSpec((1, SKV, nh * D), lambda b, h: (b, 0, h)),
            ],
            out_specs=[
                pl.BlockSpec((1, SQ, nh * D), lambda b, h: (b, 0, h)),
                pl.BlockSpec((1, nh, SQ, 1), lambda b, h: (b, h, 0, 0)),
            ],
            out_shape=[
                jax.ShapeDtypeStruct((B, SQ, H * D), jnp.float32),
                jax.ShapeDtypeStruct((B, H, SQ, 1), jnp.float32),
            ],
        )(Q2, K2, V2)
    return run


_partials_pre_h4 = _mk_partials_hblock(4)
_partials_pre_h8 = _mk_partials_hblock(8)
_partials_pre_h16 = _mk_partials_hblock(16)

_partials_pre = _mk_partials_pre(_partial_body)
_partials_pre_bf16 = _mk_partials_pre(_partial_body_bf16)
_partials_pre_kmajor = _mk_partials_pre(_partial_body_kmajor)


def _allreduce_body(x_ref, out_ref, comm_ref, send_sems, recv_sems):
    my_pos = lax.axis_index("i")
    left = (my_pos - 1) % N_DEV
    right = (my_pos + 1) % N_DEV

    barrier_sem = pltpu.get_barrier_semaphore()
    for nbr in [left, right]:
        pl.semaphore_signal(
            barrier_sem, inc=1,
            device_id=(nbr,), device_id_type=pl.DeviceIdType.MESH,
        )
    pl.semaphore_wait(barrier_sem, 2)

    comm_ref[0] = x_ref[...]
    acc = x_ref[...].astype(jnp.float32)

    for h in range(N_DEV - 1):
        send_slot = h % 2
        recv_slot = (h + 1) % 2
        rdma = pltpu.make_async_remote_copy(
            src_ref=comm_ref.at[send_slot],
            dst_ref=comm_ref.at[recv_slot],
            send_sem=send_sems.at[send_slot],
            recv_sem=recv_sems.at[recv_slot],
            device_id=(right,),
            device_id_type=pl.DeviceIdType.MESH,
        )
        rdma.start()
        rdma.wait()
        acc = acc + comm_ref[recv_slot].astype(jnp.float32)

    out_ref[...] = acc


def _ring_allreduce(buf):
    rows, n = buf.shape
    return pl.pallas_call(
        _allreduce_body,
        out_shape=jax.ShapeDtypeStruct((rows, n), jnp.float32),
        in_specs=[pl.BlockSpec(memory_space=pltpu.VMEM)],
        out_specs=pl.BlockSpec(memory_space=pltpu.VMEM),
        scratch_shapes=[
            pltpu.VMEM((2, rows, n), jnp.bfloat16),
            pltpu.SemaphoreType.DMA((2,)),
            pltpu.SemaphoreType.DMA((2,)),
        ],
        compiler_params=pltpu.CompilerParams(collective_id=0),
    )(buf)


_ROWS = SQ * H + 8


def _fused_body(qt_ref, k_ref, v_ref, out_ref, comm_ref, send_sems, recv_sems):
    b = pl.program_id(0)
    c = pl.program_id(1)
    my = lax.axis_index("i")
    left = (my - 1) % N_DEV
    right = (my + 1) % N_DEV
    diag = (my + 2) % N_DEV

    @pl.when(jnp.logical_and(b == 0, c == 0))
    def _():
        bsem = pltpu.get_barrier_semaphore()
        for nbr in [left, right, diag]:
            pl.semaphore_signal(
                bsem, inc=1,
                device_id=(nbr,), device_id_type=pl.DeviceIdType.MESH,
            )
        pl.semaphore_wait(bsem, 3)

    def rdmas_for(pb):
        out = []
        for rel, (tgt, slot) in enumerate([(right, 1), (left, 2), (diag, 3)]):
            out.append(pltpu.make_async_remote_copy(
                src_ref=comm_ref.at[pb, 0],
                dst_ref=comm_ref.at[pb, slot],
                send_sem=send_sems.at[pb, rel],
                recv_sem=recv_sems.at[pb, rel],
                device_id=(tgt,),
                device_id_type=pl.DeviceIdType.MESH,
            ))
        return out

    @pl.when(jnp.logical_and(c == 0, jnp.logical_and(b >= 1, b <= B)))
    def _():
        for r in rdmas_for(jnp.clip(b - 1, 0, B - 1)):
            r.start()

    @pl.when(jnp.logical_and(c == 0, b >= 2))
    def _():
        pb = jnp.clip(b - 2, 0, B - 1)
        rs = rdmas_for(pb)
        for r in rs:
            r.wait_recv()
        for r in rs:
            r.wait_send()
        out_ref[pb] = (
            comm_ref[pb, 0] + comm_ref[pb, 1]
            + comm_ref[pb, 2] + comm_ref[pb, 3]
        )

    @pl.when(b < B)
    def _():
        bb = jnp.clip(b, 0, B - 1)
        k = k_ref[0].astype(jnp.bfloat16)
        v = v_ref[0].astype(jnp.bfloat16)
        qt = qt_ref[0].astype(jnp.bfloat16)
        g = lax.dot_general(
            k, qt, (((1,), (0,)), ((), ())),
            preferred_element_type=jnp.float32,
        )
        rh = lax.broadcasted_iota(jnp.int32, g.shape, 0) % H
        ch = lax.broadcasted_iota(jnp.int32, g.shape, 1) % H
        p = jnp.where(rh == ch, jnp.exp(g), 0.0)
        lsum = jnp.sum(p, axis=0, keepdims=True)
        o = lax.dot_general(
            p.astype(jnp.bfloat16), v,
            (((0,), (0,)), ((), ())), preferred_element_type=jnp.float32,
        )

        @pl.when(c == 0)
        def _():
            comm_ref[bb, 0, pl.ds(0, SQ * H), :] = o
            comm_ref[bb, 0, pl.ds(SQ * H, 1), :] = lsum
            comm_ref[bb, 0, pl.ds(SQ * H + 1, _ROWS - SQ * H - 1), :] = (
                jnp.zeros((_ROWS - SQ * H - 1, D), jnp.float32)
            )

        @pl.when(c != 0)
        def _():
            comm_ref[bb, 0, pl.ds(0, SQ * H), :] += o
            comm_ref[bb, 0, pl.ds(SQ * H, 1), :] += lsum


def _fused(Q, K, V):
    Kf = K.reshape(B, SKV * H, D)
    Vf = V.reshape(B, SKV * H, D)
    ch = SKV * H // _BD_NCHUNK
    Qt = (Q * SCALE).transpose(0, 3, 1, 2).reshape(B, D, SQ * H)

    def kv_map(b, c):
        return (
            jnp.minimum(b, B - 1),
            jnp.where(b >= B, _BD_NCHUNK - 1, c),
            0,
        )

    return pl.pallas_call(
        _fused_body,
        grid=(B + 2, _BD_NCHUNK),
        in_specs=[
            pl.BlockSpec((1, D, SQ * H), lambda b, c: (jnp.minimum(b, B - 1), 0, 0)),
            pl.BlockSpec((1, ch, D), kv_map),
            pl.BlockSpec((1, ch, D), kv_map),
        ],
        out_specs=pl.BlockSpec(memory_space=pltpu.VMEM),
        out_shape=jax.ShapeDtypeStruct((B, _ROWS, D), jnp.float32),
        scratch_shapes=[
            pltpu.VMEM((B, 4, _ROWS, D), jnp.float32),
            pltpu.SemaphoreType.DMA((B, 3)),
            pltpu.SemaphoreType.DMA((B, 3)),
        ],
        compiler_params=pltpu.CompilerParams(collective_id=0),
    )(Qt, Kf, Vf)


def kernel(Q, K, V):
    red = _fused(Q, K, V)
    o_sum = red[:, : SQ * H, :].reshape(B, SQ, H, D)
    l_sum = red[:, SQ * H, :].reshape(B, SQ, H)
    return o_sum / l_sum[..., None]


def kernel_unfused(Q, K, V):
    o_part, l_part = _partials_bd(Q, K, V)
    buf = jnp.concatenate(
        [o_part.reshape(B * SQ * H, D), l_part.reshape(B, SQ * H)],
        axis=0,
    ).astype(jnp.bfloat16)
    red = _ring_allreduce(buf)
    o_sum = red[: B * SQ * H].reshape(B, SQ, H, D)
    l_sum = red[B * SQ * H :].reshape(B, SQ, H)
    return o_sum / l_sum[..., None]


# device time: 53987 ns/iter; 5.0575x vs baseline; 1.1867x over previous
import functools

import jax
import jax.numpy as jnp
from jax import lax
from jax.experimental import pallas as pl
from jax.experimental.pallas import tpu as pltpu

N_DEV = 4
B, SQ, H, D = 8, 8, 16, 128
SKV = 1024
SCALE = D ** -0.5


def _partial_body(q_ref, k_ref, v_ref, o_ref, l_ref):
    q = q_ref[0]
    k = k_ref[0]
    v = v_ref[0]
    s = lax.dot_general(
        q, k, (((1,), (1,)), ((), ())), preferred_element_type=jnp.float32
    ) * SCALE
    p = jnp.exp(s)
    l_ref[...] = jnp.sum(p, axis=1).reshape(1, 1, SQ, 1)
    o = lax.dot_general(
        p, v, (((1,), (0,)), ((), ())), preferred_element_type=jnp.float32
    )
    o_ref[0] = o


def _partials(Q, K, V):
    Q2 = Q.reshape(B, SQ, H * D)
    K2 = K.reshape(B, SKV, H * D)
    V2 = V.reshape(B, SKV, H * D)
    o, l = pl.pallas_call(
        _partial_body,
        grid=(B, H),
        in_specs=[
            pl.BlockSpec((1, SQ, D), lambda b, h: (b, 0, h)),
            pl.BlockSpec((1, SKV, D), lambda b, h: (b, 0, h)),
            pl.BlockSpec((1, SKV, D), lambda b, h: (b, 0, h)),
        ],
        out_specs=[
            pl.BlockSpec((1, SQ, D), lambda b, h: (b, 0, h)),
            pl.BlockSpec((1, 1, SQ, 1), lambda b, h: (b, h, 0, 0)),
        ],
        out_shape=[
            jax.ShapeDtypeStruct((B, SQ, H * D), jnp.float32),
            jax.ShapeDtypeStruct((B, H, SQ, 1), jnp.float32),
        ],
    )(Q2, K2, V2)
    return o, l


def _mk_partials_pre(body):
    def run(Q2, K2, V2):
        return pl.pallas_call(
            body,
            grid=(B, H),
            in_specs=[
                pl.BlockSpec((1, SQ, D), lambda b, h: (b, 0, h)),
                pl.BlockSpec((1, SKV, D), lambda b, h: (b, 0, h)),
                pl.BlockSpec((1, SKV, D), lambda b, h: (b, 0, h)),
            ],
            out_specs=[
                pl.BlockSpec((1, SQ, D), lambda b, h: (b, 0, h)),
                pl.BlockSpec((1, 1, SQ, 1), lambda b, h: (b, h, 0, 0)),
            ],
            out_shape=[
                jax.ShapeDtypeStruct((B, SQ, H * D), jnp.float32),
                jax.ShapeDtypeStruct((B, H, SQ, 1), jnp.float32),
            ],
        )(Q2, K2, V2)
    return run


def _partial_body_bf16(q_ref, k_ref, v_ref, o_ref, l_ref):
    q = q_ref[0].astype(jnp.bfloat16)
    k = k_ref[0].astype(jnp.bfloat16)
    v = v_ref[0].astype(jnp.bfloat16)
    s = lax.dot_general(
        q, k, (((1,), (1,)), ((), ())), preferred_element_type=jnp.float32
    ) * SCALE
    p = jnp.exp(s)
    l_ref[...] = jnp.sum(p, axis=1).reshape(1, 1, SQ, 1)
    o = lax.dot_general(
        p.astype(jnp.bfloat16), v,
        (((1,), (0,)), ((), ())), preferred_element_type=jnp.float32,
    )
    o_ref[0] = o


def _partial_body_kmajor(q_ref, k_ref, v_ref, o_ref, l_ref):
    q = q_ref[0]
    k = k_ref[0]
    v = v_ref[0]
    st = lax.dot_general(
        k, q, (((1,), (1,)), ((), ())), preferred_element_type=jnp.float32
    ) * SCALE
    pt = jnp.exp(st)
    l_ref[...] = jnp.sum(pt, axis=0).reshape(1, 1, SQ, 1)
    o = lax.dot_general(
        pt, v, (((0,), (0,)), ((), ())), preferred_element_type=jnp.float32
    )
    o_ref[0] = o


def _bd_body(qt_ref, k_ref, v_ref, o_ref, l_ref):
    kvc = pl.program_id(1)
    k = k_ref[0].astype(jnp.bfloat16)
    v = v_ref[0].astype(jnp.bfloat16)
    qt = qt_ref[0].astype(jnp.bfloat16)
    g = lax.dot_general(
        k, qt, (((1,), (0,)), ((), ())), preferred_element_type=jnp.float32
    )
    rh = lax.broadcasted_iota(jnp.int32, g.shape, 0) % H
    ch = lax.broadcasted_iota(jnp.int32, g.shape, 1) % H
    p = jnp.where(rh == ch, jnp.exp(g), 0.0)
    lsum = jnp.sum(p, axis=0, keepdims=True)
    o = lax.dot_general(
        p.astype(jnp.bfloat16), v,
        (((0,), (0,)), ((), ())), preferred_element_type=jnp.float32,
    )

    @pl.when(kvc == 0)
    def _():
        o_ref[0] = o
        l_ref[0] = lsum

    @pl.when(kvc != 0)
    def _():
        o_ref[0] += o
        l_ref[0] += lsum


def _mk_bd_probe(mode):
    def body(qt_ref, k_ref, v_ref, o_ref, l_ref):
        kvc = pl.program_id(1)
        k = k_ref[0].astype(jnp.bfloat16)
        v = v_ref[0].astype(jnp.bfloat16)
        qt = qt_ref[0].astype(jnp.bfloat16)
        if mode == "dma":
            o = (k[: SQ * H, :] + v[: SQ * H, :]).astype(jnp.float32)
            lsum = o[:1, :]
        else:
            g = lax.dot_general(
                k, qt, (((1,), (0,)), ((), ())),
                preferred_element_type=jnp.float32,
            )
            if mode == "noexp":
                p = g
            elif mode == "nomask":
                p = jnp.exp(g)
            else:
                rh = lax.broadcasted_iota(jnp.int32, g.shape, 0) % H
                chh = lax.broadcasted_iota(jnp.int32, g.shape, 1) % H
                p = jnp.where(rh == chh, jnp.exp(g), 0.0)
            lsum = jnp.sum(p, axis=0, keepdims=True)
            o = lax.dot_general(
                p.astype(jnp.bfloat16), v,
                (((0,), (0,)), ((), ())), preferred_element_type=jnp.float32,
            )

        @pl.when(kvc == 0)
        def _():
            o_ref[0] = o
            l_ref[0] = lsum

        @pl.when(kvc != 0)
        def _():
            o_ref[0] += o
            l_ref[0] += lsum

    def run(Q, K, V):
        Kf = K.reshape(B, SKV * H, D)
        Vf = V.reshape(B, SKV * H, D)
        chunk = SKV * H // _BD_NCHUNK
        Qt = (Q * SCALE).transpose(0, 3, 1, 2).reshape(B, D, SQ * H)
        return pl.pallas_call(
            body,
            grid=(B, _BD_NCHUNK),
            in_specs=[
                pl.BlockSpec((1, D, SQ * H), lambda b, c: (b, 0, 0)),
                pl.BlockSpec((1, chunk, D), lambda b, c: (b, c, 0)),
                pl.BlockSpec((1, chunk, D), lambda b, c: (b, c, 0)),
            ],
            out_specs=[
                pl.BlockSpec((1, SQ * H, D), lambda b, c: (b, 0, 0)),
                pl.BlockSpec((1, 1, SQ * H), lambda b, c: (b, 0, 0)),
            ],
            out_shape=[
                jax.ShapeDtypeStruct((B, SQ * H, D), jnp.float32),
                jax.ShapeDtypeStruct((B, 1, SQ * H), jnp.float32),
            ],
        )(Qt, Kf, Vf)
    return run


_probe_dma = _mk_bd_probe("dma")
_probe_noexp = _mk_bd_probe("noexp")
_probe_nomask = _mk_bd_probe("nomask")


_BD_NCHUNK = 2


def _partials_bd(Q, K, V):
    Kf = K.reshape(B, SKV * H, D)
    Vf = V.reshape(B, SKV * H, D)
    ch = SKV * H // _BD_NCHUNK
    Qt = (Q * SCALE).transpose(0, 3, 1, 2).reshape(B, D, SQ * H)
    o, l = pl.pallas_call(
        _bd_body,
        grid=(B, _BD_NCHUNK),
        in_specs=[
            pl.BlockSpec((1, D, SQ * H), lambda b, c: (b, 0, 0)),
            pl.BlockSpec((1, ch, D), lambda b, c: (b, c, 0)),
            pl.BlockSpec((1, ch, D), lambda b, c: (b, c, 0)),
        ],
        out_specs=[
            pl.BlockSpec((1, SQ * H, D), lambda b, c: (b, 0, 0)),
            pl.BlockSpec((1, 1, SQ * H), lambda b, c: (b, 0, 0)),
        ],
        out_shape=[
            jax.ShapeDtypeStruct((B, SQ * H, D), jnp.float32),
            jax.ShapeDtypeStruct((B, 1, SQ * H), jnp.float32),
        ],
    )(Qt, Kf, Vf)
    return o, l


def _mk_partials_hblock(nh):
    def body(q_ref, k_ref, v_ref, o_ref, l_ref):
        for hi in range(nh):
            sl = slice(hi * D, (hi + 1) * D)
            q = q_ref[0, :, sl]
            k = k_ref[0, :, sl]
            v = v_ref[0, :, sl]
            s = lax.dot_general(
                q, k, (((1,), (1,)), ((), ())),
                preferred_element_type=jnp.float32,
            ) * SCALE
            p = jnp.exp(s)
            l_ref[0, hi] = jnp.sum(p, axis=1, keepdims=True)
            o_ref[0, :, sl] = lax.dot_general(
                p, v, (((1,), (0,)), ((), ())),
                preferred_element_type=jnp.float32,
            )

    def run(Q2, K2, V2):
        return pl.pallas_call(
            body,
            grid=(B, H // nh),
            in_specs=[
                pl.BlockSpec((1, SQ, nh * D), lambda b, h: (b, 0, h)),
                pl.BlockSpec((1, SKV, nh * D), lambda b, h: (b, 0, h)),
                pl.BlockSpec((1, SKV, nh * D), lambda b, h: (b, 0, h)),
            ],
            out_specs=[
                pl.BlockSpec((1, SQ, nh * D), lambda b, h: (b, 0, h)),
                pl.BlockSpec((1, nh, SQ, 1), lambda b, h: (b, h, 0, 0)),
            ],
            out_shape=[
                jax.ShapeDtypeStruct((B, SQ, H * D), jnp.float32),
                jax.ShapeDtypeStruct((B, H, SQ, 1), jnp.float32),
            ],
        )(Q2, K2, V2)
    return run


_partials_pre_h4 = _mk_partials_hblock(4)
_partials_pre_h8 = _mk_partials_hblock(8)
_partials_pre_h16 = _mk_partials_hblock(16)

_partials_pre = _mk_partials_pre(_partial_body)
_partials_pre_bf16 = _mk_partials_pre(_partial_body_bf16)
_partials_pre_kmajor = _mk_partials_pre(_partial_body_kmajor)


def _allreduce_body(x_ref, out_ref, comm_ref, send_sems, recv_sems):
    my_pos = lax.axis_index("i")
    left = (my_pos - 1) % N_DEV
    right = (my_pos + 1) % N_DEV

    barrier_sem = pltpu.get_barrier_semaphore()
    for nbr in [left, right]:
        pl.semaphore_signal(
            barrier_sem, inc=1,
            device_id=(nbr,), device_id_type=pl.DeviceIdType.MESH,
        )
    pl.semaphore_wait(barrier_sem, 2)

    comm_ref[0] = x_ref[...]
    acc = x_ref[...].astype(jnp.float32)

    for h in range(N_DEV - 1):
        send_slot = h % 2
        recv_slot = (h + 1) % 2
        rdma = pltpu.make_async_remote_copy(
            src_ref=comm_ref.at[send_slot],
            dst_ref=comm_ref.at[recv_slot],
            send_sem=send_sems.at[send_slot],
            recv_sem=recv_sems.at[recv_slot],
            device_id=(right,),
            device_id_type=pl.DeviceIdType.MESH,
        )
        rdma.start()
        rdma.wait()
        acc = acc + comm_ref[recv_slot].astype(jnp.float32)

    out_ref[...] = acc


def _ring_allreduce(buf):
    rows, n = buf.shape
    return pl.pallas_call(
        _allreduce_body,
        out_shape=jax.ShapeDtypeStruct((rows, n), jnp.float32),
        in_specs=[pl.BlockSpec(memory_space=pltpu.VMEM)],
        out_specs=pl.BlockSpec(memory_space=pltpu.VMEM),
        scratch_shapes=[
            pltpu.VMEM((2, rows, n), jnp.bfloat16),
            pltpu.SemaphoreType.DMA((2,)),
            pltpu.SemaphoreType.DMA((2,)),
        ],
        compiler_params=pltpu.CompilerParams(collective_id=0),
    )(buf)


_ROWS = SQ * H + 8


def _fused_body(qt_ref, k_ref, v_ref, out_ref, comm_ref, send_sems, recv_sems):
    b = pl.program_id(0)
    c = pl.program_id(1)
    my = lax.axis_index("i")
    left = (my - 1) % N_DEV
    right = (my + 1) % N_DEV
    diag = (my + 2) % N_DEV

    @pl.when(jnp.logical_and(b == 0, c == 0))
    def _():
        bsem = pltpu.get_barrier_semaphore()
        for nbr in [left, right, diag]:
            pl.semaphore_signal(
                bsem, inc=1,
                device_id=(nbr,), device_id_type=pl.DeviceIdType.MESH,
            )
        pl.semaphore_wait(bsem, 3)

    def rdmas_for(pb):
        out = []
        for rel, (tgt, slot) in enumerate([(right, 1), (left, 2), (diag, 3)]):
            out.append(pltpu.make_async_remote_copy(
                src_ref=comm_ref.at[pb, 0],
                dst_ref=comm_ref.at[pb, slot],
                send_sem=send_sems.at[pb, rel],
                recv_sem=recv_sems.at[pb, rel],
                device_id=(tgt,),
                device_id_type=pl.DeviceIdType.MESH,
            ))
        return out

    @pl.when(jnp.logical_and(c == 0, jnp.logical_and(b >= 1, b <= B)))
    def _():
        for r in rdmas_for(jnp.clip(b - 1, 0, B - 1)):
            r.start()

    @pl.when(jnp.logical_and(c == 0, b >= 2))
    def _():
        pb = jnp.clip(b - 2, 0, B - 1)
        rs = rdmas_for(pb)
        for r in rs:
            r.wait_recv()
        for r in rs:
            r.wait_send()
        out_ref[pb] = (
            comm_ref[pb, 0] + comm_ref[pb, 1]
            + comm_ref[pb, 2] + comm_ref[pb, 3]
        )

    @pl.when(b < B)
    def _():
        bb = jnp.clip(b, 0, B - 1)
        k = k_ref[0].astype(jnp.bfloat16)
        v = v_ref[0].astype(jnp.bfloat16)
        qt = qt_ref[0].astype(jnp.bfloat16)
        g = lax.dot_general(
            k, qt, (((1,), (0,)), ((), ())),
            preferred_element_type=jnp.float32,
        )
        rh = lax.broadcasted_iota(jnp.int32, g.shape, 0) % H
        ch = lax.broadcasted_iota(jnp.int32, g.shape, 1) % H
        p = jnp.where(rh == ch, jnp.exp(g), 0.0)
        lsum = jnp.sum(p, axis=0, keepdims=True)
        o = lax.dot_general(
            p.astype(jnp.bfloat16), v,
            (((0,), (0,)), ((), ())), preferred_element_type=jnp.float32,
        )

        @pl.when(c == 0)
        def _():
            comm_ref[bb, 0, pl.ds(0, SQ * H), :] = o
            comm_ref[bb, 0, pl.ds(SQ * H, 1), :] = lsum
            comm_ref[bb, 0, pl.ds(SQ * H + 1, _ROWS - SQ * H - 1), :] = (
                jnp.zeros((_ROWS - SQ * H - 1, D), jnp.float32)
            )

        @pl.when(c != 0)
        def _():
            comm_ref[bb, 0, pl.ds(0, SQ * H), :] += o
            comm_ref[bb, 0, pl.ds(SQ * H, 1), :] += lsum


def _fused(Q, K, V):
    Kf = K.reshape(B, SKV * H, D)
    Vf = V.reshape(B, SKV * H, D)
    ch = SKV * H // _BD_NCHUNK
    Qt = (Q * SCALE).transpose(0, 3, 1, 2).reshape(B, D, SQ * H)

    def kv_map(b, c):
        return (
            jnp.minimum(b, B - 1),
            jnp.where(b >= B, _BD_NCHUNK - 1, c),
            0,
        )

    return pl.pallas_call(
        _fused_body,
        grid=(B + 2, _BD_NCHUNK),
        in_specs=[
            pl.BlockSpec((1, D, SQ * H), lambda b, c: (jnp.minimum(b, B - 1), 0, 0)),
            pl.BlockSpec((1, ch, D), kv_map),
            pl.BlockSpec((1, ch, D), kv_map),
        ],
        out_specs=pl.BlockSpec(memory_space=pltpu.VMEM),
        out_shape=jax.ShapeDtypeStruct((B, _ROWS, D), jnp.float32),
        scratch_shapes=[
            pltpu.VMEM((B, 4, _ROWS, D), jnp.float32),
            pltpu.SemaphoreType.DMA((B, 3)),
            pltpu.SemaphoreType.DMA((B, 3)),
        ],
        compiler_params=pltpu.CompilerParams(collective_id=0),
    )(Qt, Kf, Vf)


def kernel(Q, K, V):
    red = _fused(Q, K, V)
    o_sum = red[:, : SQ * H, :].reshape(B, SQ, H, D)
    l_sum = red[:, SQ * H, :].reshape(B, SQ, H)
    return o_sum / l_sum[..., None]


def kernel_unfused(Q, K, V):
    o_part, l_part = _partials_bd(Q, K, V)
    buf = jnp.concatenate(
        [o_part.reshape(B * SQ * H, D), l_part.reshape(B, SQ * H)],
        axis=0,
    ).astype(jnp.bfloat16)
    red = _ring_allreduce(buf)
    o_sum = red[: B * SQ * H].reshape(B, SQ, H, D)
    l_sum = red[B * SQ * H :].reshape(B, SQ, H)
    return o_sum / l_sum[..., None]
